# baseline (speedup 1.0000x reference)
import functools

import jax
import jax.numpy as jnp
from jax import lax
from jax.experimental import pallas as pl
from jax.experimental.pallas import tpu as pltpu
from jax.experimental.pallas import tpu_sc as plsc

LANES = 16
EPS = 1e-6


def _rsqrt_scalar(v):
    i = lax.bitcast_convert_type(v, jnp.int32)
    i = jnp.int32(0x5F3759DF) - lax.shift_right_logical(i, 1)
    y = lax.bitcast_convert_type(i, jnp.float32)
    for _ in range(2):
        y = y * (1.5 - 0.5 * v * y * y)
    return y


@functools.lru_cache(maxsize=None)
def _build(ntok, hidden, tvocab, chunk):
    info = plsc.get_sparse_core_info()
    nw = info.num_cores * info.num_subcores
    assert ntok % (nw * chunk) == 0
    tpw = ntok // nw
    nchunks = tpw // chunk
    dchunks = hidden // LANES
    inv_h = 1.0 / hidden
    mesh = plsc.VectorSubcoreMesh(core_axis_name="c", subcore_axis_name="s")

    @functools.partial(
        pl.kernel,
        out_type=jax.ShapeDtypeStruct((ntok, hidden), jnp.float32),
        mesh=mesh,
        scratch_types=[
            pltpu.VMEM((2, chunk), jnp.int32),
            pltpu.VMEM((2, chunk), jnp.int32),
            pltpu.VMEM((2, chunk + LANES), jnp.int32),
            pltpu.VMEM((chunk, hidden), jnp.float32),
            pltpu.VMEM((chunk, hidden), jnp.float32),
            pltpu.VMEM((chunk, hidden), jnp.float32),
            pltpu.VMEM((chunk, hidden), jnp.float32),
            pltpu.VMEM((tvocab, hidden), jnp.float32),
            pltpu.VMEM((hidden,), jnp.float32),
            pltpu.VMEM((hidden,), jnp.float32),
            pltpu.SemaphoreType.DMA,
            pltpu.SemaphoreType.DMA,
            pltpu.SemaphoreType.DMA,
            pltpu.SemaphoreType.DMA,
            pltpu.SemaphoreType.DMA,
            pltpu.SemaphoreType.DMA,
        ],
    )
    def sc_kernel(ids_hbm, pids_hbm, tids_hbm, wtab_hbm, ptab_hbm, ttab_hbm,
                  lnw_hbm, lnb_hbm, out_hbm,
                  widx_v, pidx_v, tids_v, wbuf0, wbuf1, pbuf0, pbuf1,
                  ttab_v, lnw_v, lnb_v,
                  sem_w0, sem_w1, sem_p0, sem_p1, sem_o0, sem_o1):
        wid = lax.axis_index("s") * info.num_cores + lax.axis_index("c")
        base = wid * tpw
        wbufs = (wbuf0, wbuf1)
        pbufs = (pbuf0, pbuf1)
        wsems = (sem_w0, sem_w1)
        psems = (sem_p0, sem_p1)
        osems = (sem_o0, sem_o1)

        pltpu.sync_copy(ttab_hbm, ttab_v)
        pltpu.sync_copy(lnw_hbm, lnw_v)
        pltpu.sync_copy(lnb_hbm, lnb_v)
        iota = lax.iota(jnp.int32, LANES)
        perms = [jnp.bitwise_xor(iota, sh)[:, None] for sh in (8, 4, 2, 1)]
        gdn = lax.GatherDimensionNumbers(
            offset_dims=(), collapsed_slice_dims=(0,), start_index_map=(0,))

        def _perm(v, pidx):
            return lax.gather(
                v, pidx, dimension_numbers=gdn, slice_sizes=(1,),
                mode=lax.GatherScatterMode.PROMISE_IN_BOUNDS)

        def lane_sums(vs):
            for pidx in perms:
                vs = [v + _perm(v, pidx) for v in vs]
            return [v[0] for v in vs]

        def idx_load(c, p):
            sl = pl.ds(base + c * chunk, chunk)
            pltpu.sync_copy(ids_hbm.at[sl], widx_v.at[p])
            pltpu.sync_copy(pids_hbm.at[sl], pidx_v.at[p])
            pltpu.sync_copy(tids_hbm.at[sl], tids_v.at[p, pl.ds(0, chunk)])

        def start_gathers(p):
            gw = pltpu.async_copy(
                wtab_hbm.at[widx_v.at[p]], wbufs[p], wsems[p])
            gp = pltpu.async_copy(
                ptab_hbm.at[pidx_v.at[p]], pbufs[p], psems[p])
            return gw, gp

        def start_out(c, p):
            return pltpu.async_copy(
                wbufs[p], out_hbm.at[pl.ds(base + c * chunk, chunk)],
                osems[p])

        def compute_chunk(p):
            wbuf = wbufs[p]
            pbuf = pbufs[p]
            tids = tids_v.at[p]
            ntok_g = 4

            def group_body(k, tcarry):
                i0 = ntok_g * k
                toks = [i0 + j for j in range(ntok_g)]
                tvec = tids[pl.ds(i0, LANES)]
                ts = [tvec[j] for j in range(ntok_g)]
                z = jnp.zeros((LANES,), jnp.float32)

                def acc_body(d, accs):
                    sl = pl.ds(d * LANES, LANES)
                    out = []
                    for j in range(ntok_g):
                        x = wbuf[toks[j], sl] + pbuf[toks[j], sl] \
                            + ttab_v[ts[j], sl]
                        wbuf[toks[j], sl] = x
                        out.append(accs[2 * j] + x)
                        out.append(accs[2 * j + 1] + x * x)
                    return tuple(out)
                accs = lax.fori_loop(
                    0, dchunks, acc_body, (z,) * (2 * ntok_g), unroll=6)
                sums = lane_sums(list(accs))
                means = [sums[2 * j] * inv_h for j in range(ntok_g)]
                rs = [_rsqrt_scalar(sums[2 * j + 1] * inv_h
                                    - means[j] * means[j] + EPS)
                      for j in range(ntok_g)]

                def norm_body(d, ncarry):
                    sl = pl.ds(d * LANES, LANES)
                    w = lnw_v[sl]
                    bi = lnb_v[sl]
                    for j in range(ntok_g):
                        y = (wbuf[toks[j], sl] - means[j]) * (rs[j] * w) + bi
                        wbuf[toks[j], sl] = y
                    return ncarry
                lax.fori_loop(0, dchunks, norm_body, 0, unroll=6)
                return tcarry
            lax.fori_loop(0, chunk // ntok_g, group_body, 0)

        idx_load(0, 0)
        gw, gp = start_gathers(0)
        gw.wait()
        gp.wait()

        out_dmas = [None, None]
        for c in range(nchunks):
            p = c % 2
            q = 1 - p
            if c + 1 < nchunks:
                if out_dmas[q] is not None:
                    out_dmas[q].wait()
                    out_dmas[q] = None
                idx_load(c + 1, q)
                gw, gp = start_gathers(q)
            compute_chunk(p)
            out_dmas[p] = start_out(c, p)
            if c + 1 < nchunks:
                gw.wait()
                gp.wait()
        for dma in out_dmas:
            if dma is not None:
                dma.wait()

    return sc_kernel


def kernel(input_ids, token_type_ids, position_ids, attention_mask,
           word_emb, pos_emb, type_emb, ln_weight, ln_bias):
    b, s = input_ids.shape
    ntok = b * s
    hidden = word_emb.shape[1]
    ids = input_ids.reshape(ntok).astype(jnp.int32)
    pids = position_ids.reshape(ntok).astype(jnp.int32)
    tids = token_type_ids.reshape(ntok).astype(jnp.int32)
    fn = _build(ntok, hidden, type_emb.shape[0], 32)
    out = fn(ids, pids, tids, word_emb, pos_emb, type_emb,
             ln_weight, ln_bias)
    return out.reshape(b, s, hidden)


# speedup vs baseline: 1.2322x; 1.2322x over previous
import functools

import jax
import jax.numpy as jnp
from jax import lax
from jax.experimental import pallas as pl
from jax.experimental.pallas import tpu as pltpu
from jax.experimental.pallas import tpu_sc as plsc

LANES = 16
EPS = 1e-6


@functools.lru_cache(maxsize=None)
def _build_sc(ntok, hidden, chunk):
    info = plsc.get_sparse_core_info()
    nw = info.num_cores * info.num_subcores
    assert ntok % (nw * chunk) == 0
    tpw = ntok // nw
    nchunks = tpw // chunk
    dchunks = hidden // LANES
    mesh = plsc.VectorSubcoreMesh(core_axis_name="c", subcore_axis_name="s")

    @functools.partial(
        pl.kernel,
        out_type=jax.ShapeDtypeStruct((ntok, hidden), jnp.float32),
        mesh=mesh,
        scratch_types=[
            pltpu.VMEM((2, chunk), jnp.int32),
            pltpu.VMEM((2, chunk), jnp.int32),
            pltpu.VMEM((chunk, hidden), jnp.float32),
            pltpu.VMEM((chunk, hidden), jnp.float32),
            pltpu.VMEM((chunk, hidden), jnp.float32),
            pltpu.VMEM((chunk, hidden), jnp.float32),
            pltpu.SemaphoreType.DMA,
            pltpu.SemaphoreType.DMA,
            pltpu.SemaphoreType.DMA,
            pltpu.SemaphoreType.DMA,
            pltpu.SemaphoreType.DMA,
            pltpu.SemaphoreType.DMA,
        ],
    )
    def sc_kernel(ids_hbm, pids_hbm, wtab_hbm, ptab_hbm, out_hbm,
                  widx_v, pidx_v, wbuf0, wbuf1, pbuf0, pbuf1,
                  sem_w0, sem_w1, sem_p0, sem_p1, sem_o0, sem_o1):
        wid = lax.axis_index("s") * info.num_cores + lax.axis_index("c")
        base = wid * tpw
        wbufs = (wbuf0, wbuf1)
        pbufs = (pbuf0, pbuf1)
        wsems = (sem_w0, sem_w1)
        psems = (sem_p0, sem_p1)
        osems = (sem_o0, sem_o1)

        def idx_load(c, p):
            sl = pl.ds(base + c * chunk, chunk)
            pltpu.sync_copy(ids_hbm.at[sl], widx_v.at[p])
            pltpu.sync_copy(pids_hbm.at[sl], pidx_v.at[p])

        def start_gathers(p):
            gw = pltpu.async_copy(
                wtab_hbm.at[widx_v.at[p]], wbufs[p], wsems[p])
            gp = pltpu.async_copy(
                ptab_hbm.at[pidx_v.at[p]], pbufs[p], psems[p])
            return gw, gp

        def start_out(c, p):
            return pltpu.async_copy(
                wbufs[p], out_hbm.at[pl.ds(base + c * chunk, chunk)],
                osems[p])

        def compute_chunk(p):
            wbuf = wbufs[p]
            pbuf = pbufs[p]

            def pair_body(k, carry):
                t0 = 2 * k
                t1 = t0 + 1

                def add_body(d, c2):
                    sl = pl.ds(d * LANES, LANES)
                    wbuf[t0, sl] = wbuf[t0, sl] + pbuf[t0, sl]
                    wbuf[t1, sl] = wbuf[t1, sl] + pbuf[t1, sl]
                    return c2
                lax.fori_loop(0, dchunks, add_body, 0, unroll=8)
                return carry
            lax.fori_loop(0, chunk // 2, pair_body, 0)

        idx_load(0, 0)
        gw, gp = start_gathers(0)
        gw.wait()
        gp.wait()

        out_dmas = [None, None]
        for c in range(nchunks):
            p = c % 2
            q = 1 - p
            if c + 1 < nchunks:
                if out_dmas[q] is not None:
                    out_dmas[q].wait()
                    out_dmas[q] = None
                idx_load(c + 1, q)
                gw, gp = start_gathers(q)
            compute_chunk(p)
            out_dmas[p] = start_out(c, p)
            if c + 1 < nchunks:
                gw.wait()
                gp.wait()
        for dma in out_dmas:
            if dma is not None:
                dma.wait()

    return sc_kernel


def _tc_ln_kernel(tvocab, x_ref, tid_ref, ttab_ref, w_ref, b_ref, o_ref):
    x = x_ref[...]
    tid = tid_ref[...]
    trow = jnp.broadcast_to(ttab_ref[0][None, :], x.shape)
    for v in range(1, tvocab):
        trow = jnp.where(tid == v, ttab_ref[v][None, :], trow)
    x = x + trow
    mean = jnp.mean(x, axis=1, keepdims=True)
    var = jnp.mean(x * x, axis=1, keepdims=True) - mean * mean
    inv = lax.rsqrt(var + EPS)
    o_ref[...] = (x - mean) * (inv * w_ref[0][None, :]) + b_ref[0][None, :]


@functools.lru_cache(maxsize=None)
def _build_tc(ntok, hidden, tvocab, bt):
    ngrid = ntok // bt
    return pl.pallas_call(
        functools.partial(_tc_ln_kernel, tvocab),
        grid=(ngrid,),
        in_specs=[
            pl.BlockSpec((bt, hidden), lambda i: (i, 0)),
            pl.BlockSpec((bt, 1), lambda i: (i, 0)),
            pl.BlockSpec((tvocab, hidden), lambda i: (0, 0)),
            pl.BlockSpec((1, hidden), lambda i: (0, 0)),
            pl.BlockSpec((1, hidden), lambda i: (0, 0)),
        ],
        out_specs=pl.BlockSpec((bt, hidden), lambda i: (i, 0)),
        out_shape=jax.ShapeDtypeStruct((ntok, hidden), jnp.float32),
    )


def kernel(input_ids, token_type_ids, position_ids, attention_mask,
           word_emb, pos_emb, type_emb, ln_weight, ln_bias):
    b, s = input_ids.shape
    ntok = b * s
    hidden = word_emb.shape[1]
    tvocab = type_emb.shape[0]
    ids = input_ids.reshape(ntok).astype(jnp.int32)
    pids = position_ids.reshape(ntok).astype(jnp.int32)
    bt = 256
    tids = token_type_ids.reshape(ntok, 1).astype(jnp.int32)
    sc = _build_sc(ntok, hidden, 32)
    summed = sc(ids, pids, word_emb, pos_emb)
    tc = _build_tc(ntok, hidden, tvocab, bt)
    out = tc(summed, tids, type_emb,
             ln_weight.reshape(1, hidden), ln_bias.reshape(1, hidden))
    return out.reshape(b, s, hidden)


# speedup vs baseline: 1.6503x; 1.3394x over previous
import functools

import jax
import jax.numpy as jnp
from jax import lax
from jax.experimental import pallas as pl
from jax.experimental.pallas import tpu as pltpu
from jax.experimental.pallas import tpu_sc as plsc

LANES = 16
EPS = 1e-6


@functools.lru_cache(maxsize=None)
def _build_sc(ntok, hidden, chunk):
    info = plsc.get_sparse_core_info()
    nw = info.num_cores * info.num_subcores
    assert ntok % (nw * chunk) == 0
    tpw = ntok // nw
    nchunks = tpw // chunk
    dchunks = hidden // LANES
    mesh = plsc.VectorSubcoreMesh(core_axis_name="c", subcore_axis_name="s")

    @functools.partial(
        pl.kernel,
        out_type=(jax.ShapeDtypeStruct((ntok, hidden), jnp.float32),
                  jax.ShapeDtypeStruct((ntok, hidden), jnp.float32)),
        mesh=mesh,
        scratch_types=[
            pltpu.VMEM((2, chunk), jnp.int32),
            pltpu.VMEM((2, chunk), jnp.int32),
            pltpu.VMEM((chunk, hidden), jnp.float32),
            pltpu.VMEM((chunk, hidden), jnp.float32),
            pltpu.VMEM((chunk, hidden), jnp.float32),
            pltpu.VMEM((chunk, hidden), jnp.float32),
            pltpu.SemaphoreType.DMA,
            pltpu.SemaphoreType.DMA,
            pltpu.SemaphoreType.DMA,
            pltpu.SemaphoreType.DMA,
            pltpu.SemaphoreType.DMA,
            pltpu.SemaphoreType.DMA,
            pltpu.SemaphoreType.DMA,
            pltpu.SemaphoreType.DMA,
        ],
    )
    def sc_kernel(ids_hbm, pids_hbm, wtab_hbm, ptab_hbm, outw_hbm, outp_hbm,
                  widx_v, pidx_v, wbuf0, wbuf1, pbuf0, pbuf1,
                  sem_w0, sem_w1, sem_p0, sem_p1,
                  sem_ow0, sem_ow1, sem_op0, sem_op1):
        wid = lax.axis_index("s") * info.num_cores + lax.axis_index("c")
        base = wid * tpw
        wbufs = (wbuf0, wbuf1)
        pbufs = (pbuf0, pbuf1)
        wsems = (sem_w0, sem_w1)
        psems = (sem_p0, sem_p1)
        owsems = (sem_ow0, sem_ow1)
        opsems = (sem_op0, sem_op1)

        def idx_load(c, p):
            sl = pl.ds(base + c * chunk, chunk)
            pltpu.sync_copy(ids_hbm.at[sl], widx_v.at[p])
            pltpu.sync_copy(pids_hbm.at[sl], pidx_v.at[p])

        def start_gathers(p):
            gw = pltpu.async_copy(
                wtab_hbm.at[widx_v.at[p]], wbufs[p], wsems[p])
            gp = pltpu.async_copy(
                ptab_hbm.at[pidx_v.at[p]], pbufs[p], psems[p])
            return gw, gp

        def start_out(c, p):
            sl = pl.ds(base + c * chunk, chunk)
            dw = pltpu.async_copy(wbufs[p], outw_hbm.at[sl], owsems[p])
            dp = pltpu.async_copy(pbufs[p], outp_hbm.at[sl], opsems[p])
            return dw, dp

        idx_load(0, 0)
        gw, gp = start_gathers(0)

        out_dmas = [None, None]
        for c in range(nchunks):
            p = c % 2
            q = 1 - p
            gw.wait()
            gp.wait()
            if c + 1 < nchunks:
                if out_dmas[q] is not None:
                    for d in out_dmas[q]:
                        d.wait()
                    out_dmas[q] = None
                idx_load(c + 1, q)
            out_dmas[p] = start_out(c, p)
            if c + 1 < nchunks:
                gw, gp = start_gathers(q)
        for dmas in out_dmas:
            if dmas is not None:
                for d in dmas:
                    d.wait()

    return sc_kernel


def _tc_ln_kernel(tvocab, xw_ref, xp_ref, tid_ref, ttab_ref, w_ref, b_ref,
                  o_ref):
    x = xw_ref[...] + xp_ref[...]
    tid = tid_ref[...]
    trow = jnp.broadcast_to(ttab_ref[0][None, :], x.shape)
    for v in range(1, tvocab):
        trow = jnp.where(tid == v, ttab_ref[v][None, :], trow)
    x = x + trow
    mean = jnp.mean(x, axis=1, keepdims=True)
    var = jnp.mean(x * x, axis=1, keepdims=True) - mean * mean
    inv = lax.rsqrt(var + EPS)
    o_ref[...] = (x - mean) * (inv * w_ref[0][None, :]) + b_ref[0][None, :]


@functools.lru_cache(maxsize=None)
def _build_tc(ntok, hidden, tvocab, bt):
    ngrid = ntok // bt
    return pl.pallas_call(
        functools.partial(_tc_ln_kernel, tvocab),
        grid=(ngrid,),
        in_specs=[
            pl.BlockSpec((bt, hidden), lambda i: (i, 0)),
            pl.BlockSpec((bt, hidden), lambda i: (i, 0)),
            pl.BlockSpec((bt, 1), lambda i: (i, 0)),
            pl.BlockSpec((tvocab, hidden), lambda i: (0, 0)),
            pl.BlockSpec((1, hidden), lambda i: (0, 0)),
            pl.BlockSpec((1, hidden), lambda i: (0, 0)),
        ],
        out_specs=pl.BlockSpec((bt, hidden), lambda i: (i, 0)),
        out_shape=jax.ShapeDtypeStruct((ntok, hidden), jnp.float32),
    )


def kernel(input_ids, token_type_ids, position_ids, attention_mask,
           word_emb, pos_emb, type_emb, ln_weight, ln_bias):
    b, s = input_ids.shape
    ntok = b * s
    hidden = word_emb.shape[1]
    tvocab = type_emb.shape[0]
    ids = input_ids.reshape(ntok).astype(jnp.int32)
    pids = position_ids.reshape(ntok).astype(jnp.int32)
    bt = 256
    tids = token_type_ids.reshape(ntok, 1).astype(jnp.int32)
    sc = _build_sc(ntok, hidden, 32)
    wrows, prows = sc(ids, pids, word_emb, pos_emb)
    tc = _build_tc(ntok, hidden, tvocab, bt)
    out = tc(wrows, prows, tids, type_emb,
             ln_weight.reshape(1, hidden), ln_bias.reshape(1, hidden))
    return out.reshape(b, s, hidden)
